# baseline (speedup 1.0000x reference)
import functools

import jax
import jax.numpy as jnp
from jax import lax
from jax.experimental import pallas as pl
from jax.experimental.pallas import tpu as pltpu
from jax.experimental.pallas import tpu_sc as plsc

NBINS = 20
LANES = 16
GRP = 32


def _make_sc_gather(V, E, B, nc, ns):
    nw = nc * ns
    rpw = B // nw
    mesh = plsc.VectorSubcoreMesh(core_axis_name="c", subcore_axis_name="s")

    @functools.partial(
        pl.kernel,
        mesh=mesh,
        compiler_params=pltpu.CompilerParams(needs_layout_passes=False),
        out_type=jax.ShapeDtypeStruct((E, B), jnp.float32),
        scratch_types=[
            pltpu.VMEM((rpw,), jnp.int32),
            pltpu.VMEM((2 * GRP * E, 128), jnp.float32),
            pltpu.VMEM((E, rpw), jnp.float32),
            pltpu.SemaphoreType.DMA,
            pltpu.SemaphoreType.DMA,
            pltpu.SemaphoreType.DMA,
        ],
    )
    def sc_gather(tab_t_hbm, idx_hbm, out_hbm, idx_v, slab_v, cols_v, sem_i,
                  sem_a, sem_b):
        wid = lax.axis_index("s") * nc + lax.axis_index("c")
        base = wid * rpw
        pltpu.async_copy(idx_hbm.at[pl.ds(base, rpw)], idx_v, sem_i).wait()
        lanes = lax.iota(jnp.int32, LANES)
        ngroups = rpw // GRP

        def fetch(g, parity):
            sem = sem_a if parity == 0 else sem_b
            for v in range(GRP // LANES):
                chunk = idx_v[pl.ds(g * GRP + v * LANES, LANES)]
                tc = lax.shift_right_logical(chunk, 7)
                for j in range(LANES):
                    off = pl.multiple_of(tc[j] * 128, 128)
                    pltpu.async_copy(
                        tab_t_hbm.at[:, pl.ds(off, 128)],
                        slab_v.at[pl.ds((parity * GRP + v * LANES + j) * E, E)],
                        sem,
                    )

        def drain(g, parity):
            sem = sem_a if parity == 0 else sem_b
            for v in range(GRP // LANES):
                chunk = idx_v[pl.ds(g * GRP + v * LANES, LANES)]
                tc = lax.shift_right_logical(chunk, 7)
                for j in range(LANES):
                    off = pl.multiple_of(tc[j] * 128, 128)
                    pltpu.make_async_copy(
                        tab_t_hbm.at[:, pl.ds(off, 128)],
                        slab_v.at[pl.ds((parity * GRP + v * LANES + j) * E, E)],
                        sem,
                    ).wait()

        def process(g, parity):
            for v in range(GRP // LANES):
                chunk = idx_v[pl.ds(g * GRP + v * LANES, LANES)]
                lane = lax.bitwise_and(chunk, 127)
                ibase = g * GRP + v * LANES + lanes
                srow = (parity * GRP + v * LANES) * E
                for c in range(E):
                    vals = plsc.load_gather(
                        slab_v, [srow + lanes * E + c, lane])
                    plsc.store_scatter(
                        cols_v, [jnp.full((LANES,), c, jnp.int32), ibase],
                        vals)

        fetch(0, 0)

        def body2(h, _):
            g0 = 2 * h
            fetch(g0 + 1, 1)
            drain(g0, 0)
            process(g0, 0)

            @pl.when(g0 + 2 < ngroups)
            def _():
                fetch(g0 + 2, 0)

            drain(g0 + 1, 1)
            process(g0 + 1, 1)
            return None

        lax.fori_loop(0, ngroups // 2, body2, None)
        pltpu.sync_copy(cols_v, out_hbm.at[:, pl.ds(base, rpw)])

    return sc_gather


def _pre_body(yr_ref, rt_ref, ytab_ref, rtab_ref, w1y_ref, w1r_ref, b1_ref,
              s_t_ref):
    f32 = jnp.float32
    yb = jnp.clip(jnp.floor(yr_ref[:] * NBINS).astype(jnp.int32), 0, NBINS - 1)
    rb = jnp.clip(jnp.floor(rt_ref[:] * NBINS).astype(jnp.int32), 0, NBINS - 1)
    iota = lax.broadcasted_iota(jnp.int32, (NBINS, 1), 0)
    oh_yt = (yb == iota).astype(f32)
    oh_rt = (rb == iota).astype(f32)
    ye_t = lax.dot_general(ytab_ref[:], oh_yt, (((0,), (0,)), ((), ())),
                           preferred_element_type=f32)
    re_t = lax.dot_general(rtab_ref[:], oh_rt, (((0,), (0,)), ((), ())),
                           preferred_element_type=f32)
    zy = lax.dot_general(w1y_ref[:], ye_t, (((0,), (0,)), ((), ())),
                         preferred_element_type=f32)
    zr = lax.dot_general(w1r_ref[:], re_t, (((0,), (0,)), ((), ())),
                         preferred_element_type=f32)
    s_t_ref[:] = zy + zr + b1_ref[:]


def _post_body(ue_t_ref, s_t_ref, w1u_ref, w2_ref, b2_ref, out_ref):
    f32 = jnp.float32
    zu = lax.dot_general(w1u_ref[:], ue_t_ref[:], (((0,), (0,)), ((), ())),
                         preferred_element_type=f32)
    h = jnp.maximum(zu + s_t_ref[:], 0.0)
    out_ref[:] = lax.dot_general(w2_ref[:], h, (((0,), (0,)), ((), ())),
                                 preferred_element_type=f32) + b2_ref[:]


def kernel(user_idx, year, num_ratings, user_table, year_table, rating_table,
           W1, b1, W2, b2):
    B = user_idx.shape[0]
    V, E = user_table.shape
    H1 = W1.shape[1]
    H2 = W2.shape[1]

    info = plsc.get_sparse_core_info()
    nc, ns = info.num_cores, info.num_subcores

    idx = user_idx.astype(jnp.int32)
    ue_t = _make_sc_gather(V, E, B, nc, ns)(user_table.T, idx)

    bm = 2048
    bm2 = 16384
    grid = (B // bm,)
    s_t = pl.pallas_call(
        _pre_body,
        grid=grid,
        in_specs=[
            pl.BlockSpec((1, bm), lambda i: (0, i)),
            pl.BlockSpec((1, bm), lambda i: (0, i)),
            pl.BlockSpec((NBINS, E), lambda i: (0, 0)),
            pl.BlockSpec((NBINS, E), lambda i: (0, 0)),
            pl.BlockSpec((E, H1), lambda i: (0, 0)),
            pl.BlockSpec((E, H1), lambda i: (0, 0)),
            pl.BlockSpec((H1, 1), lambda i: (0, 0)),
        ],
        out_specs=pl.BlockSpec((H1, bm), lambda i: (0, i)),
        out_shape=jax.ShapeDtypeStruct((H1, B), jnp.float32),
    )(
        year.reshape(1, B),
        num_ratings.reshape(1, B),
        year_table,
        rating_table,
        W1[E:2 * E],
        W1[2 * E:3 * E],
        b1.reshape(H1, 1),
    )
    out_t = pl.pallas_call(
        _post_body,
        grid=(B // bm2,),
        in_specs=[
            pl.BlockSpec((E, bm2), lambda i: (0, i)),
            pl.BlockSpec((H1, bm2), lambda i: (0, i)),
            pl.BlockSpec((E, H1), lambda i: (0, 0)),
            pl.BlockSpec((H1, H2), lambda i: (0, 0)),
            pl.BlockSpec((H2, 1), lambda i: (0, 0)),
        ],
        out_specs=pl.BlockSpec((H2, bm2), lambda i: (0, i)),
        out_shape=jax.ShapeDtypeStruct((H2, B), jnp.float32),
    )(ue_t, s_t, W1[:E], W2, b2.reshape(H2, 1))
    return out_t.T


# speedup vs baseline: 1.0395x; 1.0395x over previous
import functools

import jax
import jax.numpy as jnp
from jax import lax
from jax.experimental import pallas as pl
from jax.experimental.pallas import tpu as pltpu
from jax.experimental.pallas import tpu_sc as plsc

NBINS = 20
LANES = 16
GRP = 16
NBUF = 4


def _make_sc_gather(V, E, B, nc, ns):
    nw = nc * ns
    rpw = B // nw
    mesh = plsc.VectorSubcoreMesh(core_axis_name="c", subcore_axis_name="s")

    @functools.partial(
        pl.kernel,
        mesh=mesh,
        compiler_params=pltpu.CompilerParams(needs_layout_passes=False),
        out_type=jax.ShapeDtypeStruct((E, B), jnp.float32),
        scratch_types=[
            pltpu.VMEM((rpw,), jnp.int32),
            pltpu.VMEM((NBUF * GRP * E, 128), jnp.float32),
            pltpu.VMEM((E, rpw), jnp.float32),
            pltpu.SemaphoreType.DMA,
            pltpu.SemaphoreType.DMA,
            pltpu.SemaphoreType.DMA,
            pltpu.SemaphoreType.DMA,
            pltpu.SemaphoreType.DMA,
        ],
    )
    def sc_gather(tab_t_hbm, idx_hbm, out_hbm, idx_v, slab_v, cols_v, sem_i,
                  s0, s1, s2, s3):
        sems = (s0, s1, s2, s3)
        wid = lax.axis_index("s") * nc + lax.axis_index("c")
        base = wid * rpw
        pltpu.async_copy(idx_hbm.at[pl.ds(base, rpw)], idx_v, sem_i).wait()
        lanes = lax.iota(jnp.int32, LANES)
        ngroups = rpw // GRP

        def fetch(g, parity):
            sem = sems[parity]
            chunk = idx_v[pl.ds(g * GRP, GRP)]
            tc = lax.shift_right_logical(chunk, 7)
            for j in range(GRP):
                off = pl.multiple_of(tc[j] * 128, 128)
                pltpu.async_copy(
                    tab_t_hbm.at[:, pl.ds(off, 128)],
                    slab_v.at[pl.ds((parity * GRP + j) * E, E)],
                    sem,
                )

        def drain(g, parity):
            sem = sems[parity]
            chunk = idx_v[pl.ds(g * GRP, GRP)]
            tc = lax.shift_right_logical(chunk, 7)
            for j in range(GRP):
                off = pl.multiple_of(tc[j] * 128, 128)
                pltpu.make_async_copy(
                    tab_t_hbm.at[:, pl.ds(off, 128)],
                    slab_v.at[pl.ds((parity * GRP + j) * E, E)],
                    sem,
                ).wait()

        def process(g, parity):
            chunk = idx_v[pl.ds(g * GRP, GRP)]
            lane = lax.bitwise_and(chunk, 127)
            ibase = g * GRP + lanes
            srow = parity * GRP * E
            for c in range(E):
                vals = plsc.load_gather(slab_v, [srow + lanes * E + c, lane])
                plsc.store_scatter(
                    cols_v, [jnp.full((LANES,), c, jnp.int32), ibase], vals)

        for q in range(NBUF - 1):
            fetch(q, q)

        def body(h, _):
            g0 = NBUF * h
            for q in range(NBUF):
                g = g0 + q

                @pl.when(g + NBUF - 1 < ngroups)
                def _():
                    fetch(g + NBUF - 1, (q + NBUF - 1) % NBUF)

                drain(g, q)
                process(g, q)
            return None

        lax.fori_loop(0, ngroups // NBUF, body, None)
        pltpu.sync_copy(cols_v, out_hbm.at[:, pl.ds(base, rpw)])

    return sc_gather


def _pre_body(yr_ref, rt_ref, ytab_ref, rtab_ref, w1y_ref, w1r_ref, b1_ref,
              s_t_ref):
    f32 = jnp.float32
    yb = jnp.clip(jnp.floor(yr_ref[:] * NBINS).astype(jnp.int32), 0, NBINS - 1)
    rb = jnp.clip(jnp.floor(rt_ref[:] * NBINS).astype(jnp.int32), 0, NBINS - 1)
    iota = lax.broadcasted_iota(jnp.int32, (NBINS, 1), 0)
    oh_yt = (yb == iota).astype(f32)
    oh_rt = (rb == iota).astype(f32)
    ye_t = lax.dot_general(ytab_ref[:], oh_yt, (((0,), (0,)), ((), ())),
                           preferred_element_type=f32)
    re_t = lax.dot_general(rtab_ref[:], oh_rt, (((0,), (0,)), ((), ())),
                           preferred_element_type=f32)
    zy = lax.dot_general(w1y_ref[:], ye_t, (((0,), (0,)), ((), ())),
                         preferred_element_type=f32)
    zr = lax.dot_general(w1r_ref[:], re_t, (((0,), (0,)), ((), ())),
                         preferred_element_type=f32)
    s_t_ref[:] = zy + zr + b1_ref[:]


def _post_body(ue_t_ref, s_t_ref, w1u_ref, w2_ref, b2_ref, out_ref):
    f32 = jnp.float32
    zu = lax.dot_general(w1u_ref[:], ue_t_ref[:], (((0,), (0,)), ((), ())),
                         preferred_element_type=f32)
    h = jnp.maximum(zu + s_t_ref[:], 0.0)
    out_ref[:] = lax.dot_general(w2_ref[:], h, (((0,), (0,)), ((), ())),
                                 preferred_element_type=f32) + b2_ref[:]


def kernel(user_idx, year, num_ratings, user_table, year_table, rating_table,
           W1, b1, W2, b2):
    B = user_idx.shape[0]
    V, E = user_table.shape
    H1 = W1.shape[1]
    H2 = W2.shape[1]

    info = plsc.get_sparse_core_info()
    nc, ns = info.num_cores, info.num_subcores

    idx = user_idx.astype(jnp.int32)
    ue_t = _make_sc_gather(V, E, B, nc, ns)(user_table.T, idx)

    bm = 2048
    bm2 = 8192
    grid = (B // bm,)
    s_t = pl.pallas_call(
        _pre_body,
        grid=grid,
        in_specs=[
            pl.BlockSpec((1, bm), lambda i: (0, i)),
            pl.BlockSpec((1, bm), lambda i: (0, i)),
            pl.BlockSpec((NBINS, E), lambda i: (0, 0)),
            pl.BlockSpec((NBINS, E), lambda i: (0, 0)),
            pl.BlockSpec((E, H1), lambda i: (0, 0)),
            pl.BlockSpec((E, H1), lambda i: (0, 0)),
            pl.BlockSpec((H1, 1), lambda i: (0, 0)),
        ],
        out_specs=pl.BlockSpec((H1, bm), lambda i: (0, i)),
        out_shape=jax.ShapeDtypeStruct((H1, B), jnp.float32),
    )(
        year.reshape(1, B),
        num_ratings.reshape(1, B),
        year_table,
        rating_table,
        W1[E:2 * E],
        W1[2 * E:3 * E],
        b1.reshape(H1, 1),
    )
    out_t = pl.pallas_call(
        _post_body,
        grid=(B // bm2,),
        in_specs=[
            pl.BlockSpec((E, bm2), lambda i: (0, i)),
            pl.BlockSpec((H1, bm2), lambda i: (0, i)),
            pl.BlockSpec((E, H1), lambda i: (0, 0)),
            pl.BlockSpec((H1, H2), lambda i: (0, 0)),
            pl.BlockSpec((H2, 1), lambda i: (0, 0)),
        ],
        out_specs=pl.BlockSpec((H2, bm2), lambda i: (0, i)),
        out_shape=jax.ShapeDtypeStruct((H2, B), jnp.float32),
    )(ue_t, s_t, W1[:E], W2, b2.reshape(H2, 1))
    return out_t.T
